# baseline (speedup 1.0000x reference)
import jax
import jax.numpy as jnp
from jax import lax
from jax.experimental import pallas as pl
from jax.experimental.pallas import tpu as pltpu
from jax.experimental.pallas import tpu_sc as plsc

_NC = 2
_NS = 16
_NW = _NC * _NS
_CHUNK = 128


def _glove_body(words_hbm, contexts_hbm, w_emb_hbm, w_bias_hbm, c_emb_hbm,
                c_bias_hbm, out_we, out_wb, out_ce, out_cb,
                idx_w, idx_c, we_rows, ce_rows, wb_rows, cb_rows, sem):
    nchunk = idx_w.shape[0]
    bpw = nchunk * _CHUNK
    wid = lax.axis_index("s") * _NC + lax.axis_index("c")
    base = wid * bpw
    row0 = wid * nchunk
    pltpu.sync_copy(words_hbm.at[pl.ds(row0, nchunk)], idx_w)
    pltpu.sync_copy(contexts_hbm.at[pl.ds(row0, nchunk)], idx_c)
    copies = []
    for j in range(nchunk):
        dst = pl.ds(j * _CHUNK, _CHUNK)
        copies.append(pltpu.async_copy(w_emb_hbm.at[idx_w.at[j]],
                                       we_rows.at[dst], sem))
        copies.append(pltpu.async_copy(c_emb_hbm.at[idx_c.at[j]],
                                       ce_rows.at[dst], sem))
        copies.append(pltpu.async_copy(w_bias_hbm.at[idx_w.at[j]],
                                       wb_rows.at[dst], sem))
        copies.append(pltpu.async_copy(c_bias_hbm.at[idx_c.at[j]],
                                       cb_rows.at[dst], sem))
    for c in copies:
        c.wait()
    pltpu.sync_copy(we_rows, out_we.at[pl.ds(base, bpw)])
    pltpu.sync_copy(ce_rows, out_ce.at[pl.ds(base, bpw)])
    pltpu.sync_copy(wb_rows, out_wb.at[pl.ds(base, bpw)])
    pltpu.sync_copy(cb_rows, out_cb.at[pl.ds(base, bpw)])


def kernel(words, contexts, w_embeddings, w_biases, c_embeddings, c_biases):
    batch = words.shape[0]
    dim = w_embeddings.shape[1]
    bpw = batch // _NW
    nchunk = bpw // _CHUNK
    mesh = plsc.VectorSubcoreMesh(core_axis_name="c", subcore_axis_name="s")
    f = pl.kernel(
        _glove_body,
        out_type=(
            jax.ShapeDtypeStruct((batch, dim), jnp.float32),
            jax.ShapeDtypeStruct((batch,), jnp.float32),
            jax.ShapeDtypeStruct((batch, dim), jnp.float32),
            jax.ShapeDtypeStruct((batch,), jnp.float32),
        ),
        mesh=mesh,
        compiler_params=pltpu.CompilerParams(use_tc_tiling_on_sc=False),
        scratch_types=[
            pltpu.VMEM((nchunk, _CHUNK), jnp.int32),
            pltpu.VMEM((nchunk, _CHUNK), jnp.int32),
            pltpu.VMEM((bpw, dim), jnp.float32),
            pltpu.VMEM((bpw, dim), jnp.float32),
            pltpu.VMEM((bpw,), jnp.float32),
            pltpu.VMEM((bpw,), jnp.float32),
            pltpu.SemaphoreType.DMA,
        ],
    )
    we, wb, ce, cb = f(
        words.reshape(_NW * nchunk, _CHUNK),
        contexts.reshape(_NW * nchunk, _CHUNK),
        w_embeddings,
        w_biases.reshape(-1),
        c_embeddings,
        c_biases.reshape(-1),
    )
    return we, wb.reshape(batch, 1), ce, cb.reshape(batch, 1)


# speedup vs baseline: 1.8722x; 1.8722x over previous
import jax
import jax.numpy as jnp
from jax import lax
from jax.experimental import pallas as pl
from jax.experimental.pallas import tpu as pltpu
from jax.experimental.pallas import tpu_sc as plsc

_NC = 2
_NS = 16
_NW = _NC * _NS
_L = 128
_NBUF = 8


def _gather_table(embT, out, idx, blks, buf, sem, base, bpw):
    dim = embT.shape[0]
    nchunk = bpw // _L
    iotas = [lax.iota(jnp.int32, 16) + u * 16 for u in range(dim // 16)]

    for m in range(nchunk):
        def body(g, carry):
            k0 = m * _L + g * 16
            v16 = idx[pl.ds(k0, 16)]
            c16 = (v16 >> 7) * _L
            lane16 = v16 & 127
            for half in range(2):
                for t in range(_NBUF):
                    c_off = pl.multiple_of(c16[half * _NBUF + t], _L)
                    pltpu.async_copy(embT.at[:, pl.ds(c_off, _L)],
                                     blks[t], sem)
                for t in range(_NBUF):
                    pltpu.make_async_copy(embT.at[:, pl.ds(0, _L)],
                                          blks[t], sem).wait()
                for t in range(_NBUF):
                    tt = half * _NBUF + t
                    lane = lane16[tt]
                    kk = g * 16 + tt
                    for u in range(dim // 16):
                        buf[pl.ds(kk * dim + u * 16, 16)] = plsc.load_gather(
                            blks[t], [iotas[u], iotas[u] * 0 + lane])
            return carry

        lax.fori_loop(0, _L // 16, body, 0)
        pltpu.sync_copy(buf, out.at[pl.ds((base + m * _L) * dim, _L * dim)])


def _glove_body(words_hbm, contexts_hbm, w_embT, w_bias, c_embT, c_bias,
                out_we, out_wb, out_ce, out_cb,
                idx_w, idx_c, blk0, blk1, blk2, blk3, blk4, blk5, blk6, blk7, buf,
                wb_v, cb_v, sem_g, sem_b):
    bpw = idx_w.shape[0]
    nchunk = bpw // _L
    wid = lax.axis_index("s") * _NC + lax.axis_index("c")
    base = pl.multiple_of(wid * bpw, bpw)
    pltpu.sync_copy(words_hbm.at[pl.ds(base, bpw)], idx_w)
    pltpu.sync_copy(contexts_hbm.at[pl.ds(base, bpw)], idx_c)

    bias_copies = []
    for m in range(nchunk):
        s = pl.ds(m * _L, _L)
        bias_copies.append(
            pltpu.async_copy(w_bias.at[idx_w.at[s]], wb_v.at[s], sem_b))
        bias_copies.append(
            pltpu.async_copy(c_bias.at[idx_c.at[s]], cb_v.at[s], sem_b))

    blks = [blk0, blk1, blk2, blk3, blk4, blk5, blk6, blk7]
    _gather_table(w_embT, out_we, idx_w, blks, buf, sem_g, base, bpw)
    _gather_table(c_embT, out_ce, idx_c, blks, buf, sem_g, base, bpw)

    for c in bias_copies:
        c.wait()
    pltpu.sync_copy(wb_v, out_wb.at[pl.ds(base, bpw)])
    pltpu.sync_copy(cb_v, out_cb.at[pl.ds(base, bpw)])


def kernel(words, contexts, w_embeddings, w_biases, c_embeddings, c_biases):
    batch = words.shape[0]
    vocab, dim = w_embeddings.shape
    bpw = batch // _NW
    mesh = plsc.VectorSubcoreMesh(core_axis_name="c", subcore_axis_name="s")
    f = pl.kernel(
        _glove_body,
        out_type=(
            jax.ShapeDtypeStruct((batch * dim,), jnp.float32),
            jax.ShapeDtypeStruct((batch,), jnp.float32),
            jax.ShapeDtypeStruct((batch * dim,), jnp.float32),
            jax.ShapeDtypeStruct((batch,), jnp.float32),
        ),
        mesh=mesh,
        compiler_params=pltpu.CompilerParams(
            use_tc_tiling_on_sc=True, needs_layout_passes=False),
        scratch_types=[
            pltpu.VMEM((bpw,), jnp.int32),
            pltpu.VMEM((bpw,), jnp.int32),
            pltpu.VMEM((dim, _L), jnp.float32),
            pltpu.VMEM((dim, _L), jnp.float32),
            pltpu.VMEM((dim, _L), jnp.float32),
            pltpu.VMEM((dim, _L), jnp.float32),
            pltpu.VMEM((dim, _L), jnp.float32),
            pltpu.VMEM((dim, _L), jnp.float32),
            pltpu.VMEM((dim, _L), jnp.float32),
            pltpu.VMEM((dim, _L), jnp.float32),
            pltpu.VMEM((_L * dim,), jnp.float32),
            pltpu.VMEM((bpw,), jnp.float32),
            pltpu.VMEM((bpw,), jnp.float32),
            pltpu.SemaphoreType.DMA,
            pltpu.SemaphoreType.DMA,
        ],
    )
    we, wb, ce, cb = f(
        words, contexts,
        w_embeddings.T, w_biases.reshape(-1),
        c_embeddings.T, c_biases.reshape(-1),
    )
    return (we.reshape(batch, dim), wb.reshape(batch, 1),
            ce.reshape(batch, dim), cb.reshape(batch, 1))


# speedup vs baseline: 1.8866x; 1.0077x over previous
import jax
import jax.numpy as jnp
from jax import lax
from jax.experimental import pallas as pl
from jax.experimental.pallas import tpu as pltpu
from jax.experimental.pallas import tpu_sc as plsc

_NC = 2
_NS = 16
_NW = _NC * _NS
_L = 128
_NBUF = 8


def _gather_table(embT, out, idx, blks, buf, sem, base, bpw):
    dim = embT.shape[0]
    nchunk = bpw // _L
    iotas = [lax.iota(jnp.int32, 16) + u * 16 for u in range(dim // 16)]

    def body(g, carry):
        k0 = g * 16
        v16 = idx[pl.ds(k0, 16)]
        c16 = (v16 >> 7) * _L
        lane16 = v16 & 127
        for half in range(2):
            for t in range(_NBUF):
                c_off = pl.multiple_of(c16[half * _NBUF + t], _L)
                pltpu.async_copy(embT.at[:, pl.ds(c_off, _L)],
                                 blks[t], sem)
            for t in range(_NBUF):
                pltpu.make_async_copy(embT.at[:, pl.ds(0, _L)],
                                      blks[t], sem).wait()
            for t in range(_NBUF):
                tt = half * _NBUF + t
                lane = lane16[tt]
                kk = k0 + tt
                for u in range(dim // 16):
                    rows = plsc.load_gather(
                        blks[t], [iotas[u], iotas[u] * 0 + lane])
                    plsc.store_scatter(buf, [iotas[u], iotas[u] * 0 + kk],
                                       rows)
        return carry

    lax.fori_loop(0, bpw // 16, body, 0)
    pltpu.sync_copy(buf, out.at[:, pl.ds(base, bpw)])


def _glove_body(words_hbm, contexts_hbm, w_embT, w_bias, c_embT, c_bias,
                out_we, out_wb, out_ce, out_cb,
                idx_w, idx_c, blk0, blk1, blk2, blk3, blk4, blk5, blk6, blk7, buf,
                wb_v, cb_v, sem_g, sem_b):
    bpw = idx_w.shape[0]
    nchunk = bpw // _L
    wid = lax.axis_index("s") * _NC + lax.axis_index("c")
    base = pl.multiple_of(wid * bpw, bpw)
    pltpu.sync_copy(words_hbm.at[pl.ds(base, bpw)], idx_w)
    pltpu.sync_copy(contexts_hbm.at[pl.ds(base, bpw)], idx_c)

    bias_copies = []
    for m in range(nchunk):
        s = pl.ds(m * _L, _L)
        bias_copies.append(
            pltpu.async_copy(w_bias.at[idx_w.at[s]], wb_v.at[s], sem_b))
        bias_copies.append(
            pltpu.async_copy(c_bias.at[idx_c.at[s]], cb_v.at[s], sem_b))

    blks = [blk0, blk1, blk2, blk3, blk4, blk5, blk6, blk7]
    _gather_table(w_embT, out_we, idx_w, blks, buf, sem_g, base, bpw)
    _gather_table(c_embT, out_ce, idx_c, blks, buf, sem_g, base, bpw)

    for c in bias_copies:
        c.wait()
    pltpu.sync_copy(wb_v, out_wb.at[pl.ds(base, bpw)])
    pltpu.sync_copy(cb_v, out_cb.at[pl.ds(base, bpw)])


def kernel(words, contexts, w_embeddings, w_biases, c_embeddings, c_biases):
    batch = words.shape[0]
    vocab, dim = w_embeddings.shape
    bpw = batch // _NW
    mesh = plsc.VectorSubcoreMesh(core_axis_name="c", subcore_axis_name="s")
    f = pl.kernel(
        _glove_body,
        out_type=(
            jax.ShapeDtypeStruct((dim, batch), jnp.float32),
            jax.ShapeDtypeStruct((batch,), jnp.float32),
            jax.ShapeDtypeStruct((dim, batch), jnp.float32),
            jax.ShapeDtypeStruct((batch,), jnp.float32),
        ),
        mesh=mesh,
        compiler_params=pltpu.CompilerParams(
            use_tc_tiling_on_sc=True, needs_layout_passes=False),
        scratch_types=[
            pltpu.VMEM((bpw,), jnp.int32),
            pltpu.VMEM((bpw,), jnp.int32),
            pltpu.VMEM((dim, _L), jnp.float32),
            pltpu.VMEM((dim, _L), jnp.float32),
            pltpu.VMEM((dim, _L), jnp.float32),
            pltpu.VMEM((dim, _L), jnp.float32),
            pltpu.VMEM((dim, _L), jnp.float32),
            pltpu.VMEM((dim, _L), jnp.float32),
            pltpu.VMEM((dim, _L), jnp.float32),
            pltpu.VMEM((dim, _L), jnp.float32),
            pltpu.VMEM((dim, bpw), jnp.float32),
            pltpu.VMEM((bpw,), jnp.float32),
            pltpu.VMEM((bpw,), jnp.float32),
            pltpu.SemaphoreType.DMA,
            pltpu.SemaphoreType.DMA,
        ],
    )
    we, wb, ce, cb = f(
        words, contexts,
        w_embeddings.T, w_biases.reshape(-1),
        c_embeddings.T, c_biases.reshape(-1),
    )
    return (we.T, wb.reshape(batch, 1), ce.T, cb.reshape(batch, 1))


# speedup vs baseline: 2.4906x; 1.3202x over previous
import jax
import jax.numpy as jnp
from jax import lax
from jax.experimental import pallas as pl
from jax.experimental.pallas import tpu as pltpu
from jax.experimental.pallas import tpu_sc as plsc

_NC = 2
_NS = 16
_NW = _NC * _NS
_L = 128
_NBUF = 8


def _gather_table(embT, out, idx, vsm, blks, buf, sem, base, bpw):
    dim = embT.shape[0]
    iotas = [lax.iota(jnp.int32, 16) + u * 16 for u in range(dim // 16)]

    def stage(g, carry):
        v16 = idx[pl.ds(g * 16, 16)]
        for t in range(16):
            vsm[g * 16 + t] = v16[t]
        return carry

    lax.fori_loop(0, bpw // 16, stage, 0)

    def fire(k, blk):
        c_off = pl.multiple_of((vsm[k] >> 7) * _L, _L)
        pltpu.async_copy(embT.at[:, pl.ds(c_off, _L)], blk, sem)

    for p in range(_NBUF):
        fire(p, blks[p])

    def body(g, carry):
        for p in range(_NBUF):
            k = g * _NBUF + p
            blk = blks[p]
            pltpu.make_async_copy(embT.at[:, pl.ds(0, _L)], blk, sem).wait()
            lane = vsm[k] & 127
            for u in range(dim // 16):
                rows = plsc.load_gather(
                    blk, [iotas[u], iotas[u] * 0 + lane])
                plsc.store_scatter(buf, [iotas[u], iotas[u] * 0 + k], rows)

            @pl.when(k + _NBUF < bpw)
            def _():
                fire(k + _NBUF, blk)
        return carry

    lax.fori_loop(0, bpw // _NBUF, body, 0)
    pltpu.sync_copy(buf, out.at[:, pl.ds(base, bpw)])


def _glove_body(words_hbm, contexts_hbm, w_embT, w_bias, c_embT, c_bias,
                out_we, out_wb, out_ce, out_cb,
                idx_w, idx_c, vsm, blk0, blk1, blk2, blk3, blk4, blk5, blk6, blk7, buf,
                wb_v, cb_v, sem_g, sem_b):
    bpw = idx_w.shape[0]
    nchunk = bpw // _L
    wid = lax.axis_index("s") * _NC + lax.axis_index("c")
    base = pl.multiple_of(wid * bpw, bpw)
    pltpu.sync_copy(words_hbm.at[pl.ds(base, bpw)], idx_w)
    pltpu.sync_copy(contexts_hbm.at[pl.ds(base, bpw)], idx_c)

    bias_copies = []
    for m in range(nchunk):
        s = pl.ds(m * _L, _L)
        bias_copies.append(
            pltpu.async_copy(w_bias.at[idx_w.at[s]], wb_v.at[s], sem_b))
        bias_copies.append(
            pltpu.async_copy(c_bias.at[idx_c.at[s]], cb_v.at[s], sem_b))

    blks = [blk0, blk1, blk2, blk3, blk4, blk5, blk6, blk7]
    _gather_table(w_embT, out_we, idx_w, vsm, blks, buf, sem_g, base, bpw)
    _gather_table(c_embT, out_ce, idx_c, vsm, blks, buf, sem_g, base, bpw)

    for c in bias_copies:
        c.wait()
    pltpu.sync_copy(wb_v, out_wb.at[pl.ds(base, bpw)])
    pltpu.sync_copy(cb_v, out_cb.at[pl.ds(base, bpw)])


def kernel(words, contexts, w_embeddings, w_biases, c_embeddings, c_biases):
    batch = words.shape[0]
    vocab, dim = w_embeddings.shape
    bpw = batch // _NW
    mesh = plsc.VectorSubcoreMesh(core_axis_name="c", subcore_axis_name="s")
    f = pl.kernel(
        _glove_body,
        out_type=(
            jax.ShapeDtypeStruct((dim, batch), jnp.float32),
            jax.ShapeDtypeStruct((batch,), jnp.float32),
            jax.ShapeDtypeStruct((dim, batch), jnp.float32),
            jax.ShapeDtypeStruct((batch,), jnp.float32),
        ),
        mesh=mesh,
        compiler_params=pltpu.CompilerParams(
            use_tc_tiling_on_sc=True, needs_layout_passes=False),
        scratch_types=[
            pltpu.VMEM((bpw,), jnp.int32),
            pltpu.VMEM((bpw,), jnp.int32),
            pltpu.SMEM((bpw,), jnp.int32),
            pltpu.VMEM((dim, _L), jnp.float32),
            pltpu.VMEM((dim, _L), jnp.float32),
            pltpu.VMEM((dim, _L), jnp.float32),
            pltpu.VMEM((dim, _L), jnp.float32),
            pltpu.VMEM((dim, _L), jnp.float32),
            pltpu.VMEM((dim, _L), jnp.float32),
            pltpu.VMEM((dim, _L), jnp.float32),
            pltpu.VMEM((dim, _L), jnp.float32),
            pltpu.VMEM((dim, bpw), jnp.float32),
            pltpu.VMEM((bpw,), jnp.float32),
            pltpu.VMEM((bpw,), jnp.float32),
            pltpu.SemaphoreType.DMA,
            pltpu.SemaphoreType.DMA,
        ],
    )
    we, wb, ce, cb = f(
        words, contexts,
        w_embeddings.T, w_biases.reshape(-1),
        c_embeddings.T, c_biases.reshape(-1),
    )
    return (we.T, wb.reshape(batch, 1), ce.T, cb.reshape(batch, 1))
